# baseline (speedup 1.0000x reference)
import functools

import jax
import jax.numpy as jnp
from jax import lax
from jax.experimental import pallas as pl
from jax.experimental.pallas import tpu as pltpu
from jax.experimental.pallas import tpu_sc as plsc

_BATCH = 4096
_SEQ = 200
_D = 32
_N = _BATCH * _SEQ
_NW = 32
_PER_W = _N // _NW
_C = 800
_NCHUNK = _PER_W // _C
_NPAIR = _NCHUNK // 2

_mesh = plsc.VectorSubcoreMesh(core_axis_name="c", subcore_axis_name="s")


@functools.partial(
    pl.kernel,
    mesh=_mesh,
    out_type=jax.ShapeDtypeStruct((_N, _D), jnp.float32),
    scratch_types=[
        pltpu.VMEM((_PER_W,), jnp.int32),
        pltpu.VMEM((_PER_W,), jnp.int32),
        pltpu.VMEM((_C, _D), jnp.float32),
        pltpu.VMEM((_C, _D), jnp.float32),
        pltpu.SemaphoreType.DMA,
        pltpu.SemaphoreType.DMA,
        pltpu.SemaphoreType.DMA,
        pltpu.SemaphoreType.DMA,
        pltpu.SemaphoreType.DMA,
    ],
    compiler_params=pltpu.CompilerParams(use_tc_tiling_on_sc=False),
)
def _double_gather(ids_hbm, remap_hbm, emb_hbm, out_hbm,
                   ids_v, gids_v, r0, r1, sem_a, sg0, sg1, ss0, ss1):
    wid = lax.axis_index("s") * 2 + lax.axis_index("c")
    base = wid * _PER_W
    rows_per_w = _PER_W // _SEQ

    def ld_row(i, carry):
        pltpu.async_copy(ids_hbm.at[wid * rows_per_w + i, :],
                         ids_v.at[pl.ds(i * _SEQ, _SEQ)], sem_a)
        return carry

    lax.fori_loop(0, rows_per_w, ld_row, 0)

    def ld_row_wait(i, carry):
        pltpu.make_async_copy(ids_hbm.at[0, :], ids_v.at[pl.ds(0, _SEQ)],
                              sem_a).wait()
        return carry

    lax.fori_loop(0, rows_per_w, ld_row_wait, 0)

    pltpu.async_copy(remap_hbm.at[ids_v], gids_v, sem_a).wait()

    def g2_start(i, buf, sem):
        pltpu.async_copy(emb_hbm.at[gids_v.at[pl.ds(i * _C, _C)]], buf, sem)

    def g2_wait(buf, sem):
        pltpu.make_async_copy(emb_hbm.at[gids_v.at[pl.ds(0, _C)]], buf,
                              sem).wait()

    def st_start(i, buf, sem):
        pltpu.async_copy(buf, out_hbm.at[pl.ds(base + i * _C, _C)], sem)

    def st_wait(buf, sem):
        pltpu.make_async_copy(buf, out_hbm.at[pl.ds(base, _C)], sem).wait()

    g2_start(0, r0, sg0)

    def pair(j, carry):
        i0 = 2 * j

        @pl.when(j > 0)
        def _():
            st_wait(r1, ss1)
        g2_start(i0 + 1, r1, sg1)
        g2_wait(r0, sg0)
        st_start(i0, r0, ss0)

        @pl.when(j < _NPAIR - 1)
        def _():
            st_wait(r0, ss0)
            g2_start(i0 + 2, r0, sg0)
        g2_wait(r1, sg1)
        st_start(i0 + 1, r1, ss1)
        return carry

    lax.fori_loop(0, _NPAIR, pair, 0)
    st_wait(r0, ss0)
    st_wait(r1, ss1)


def kernel(client_ids, item_ids, item_id2graph_id, item_embeddings):
    del client_ids
    out = _double_gather(item_ids.astype(jnp.int32),
                         item_id2graph_id.astype(jnp.int32),
                         item_embeddings)
    return out.reshape(_BATCH, _SEQ, _D)


# speedup vs baseline: 1.0497x; 1.0497x over previous
import functools

import jax
import jax.numpy as jnp
from jax import lax
from jax.experimental import pallas as pl
from jax.experimental.pallas import tpu as pltpu
from jax.experimental.pallas import tpu_sc as plsc

_BATCH = 4096
_SEQ = 200
_D = 32
_N = _BATCH * _SEQ
_NW = 32
_PER_W = _N // _NW
_C = 800
_NCHUNK = _PER_W // _C
_NPAIR = _NCHUNK // 2

_mesh = plsc.VectorSubcoreMesh(core_axis_name="c", subcore_axis_name="s")


@functools.partial(
    pl.kernel,
    mesh=_mesh,
    out_type=jax.ShapeDtypeStruct((_N, _D), jnp.float32),
    scratch_types=[
        pltpu.VMEM((_PER_W,), jnp.int32),
        pltpu.VMEM((_PER_W,), jnp.int32),
        pltpu.VMEM((_C, _D), jnp.float32),
        pltpu.VMEM((_C, _D), jnp.float32),
        pltpu.SemaphoreType.DMA,
        pltpu.SemaphoreType.DMA,
        pltpu.SemaphoreType.DMA,
        pltpu.SemaphoreType.DMA,
        pltpu.SemaphoreType.DMA,
    ],
    compiler_params=pltpu.CompilerParams(use_tc_tiling_on_sc=False),
)
def _double_gather(ids_hbm, remap_hbm, emb_hbm, out_hbm,
                   ids_v, gids_v, r0, r1, sem_a, sg0, sg1, ss0, ss1):
    wid = lax.axis_index("s") * 2 + lax.axis_index("c")
    base = wid * _PER_W

    pltpu.sync_copy(ids_hbm.at[pl.ds(base, _PER_W)], ids_v)

    pltpu.async_copy(remap_hbm.at[ids_v], gids_v, sem_a).wait()

    def g2_start(i, buf, sem):
        pltpu.async_copy(emb_hbm.at[gids_v.at[pl.ds(i * _C, _C)]], buf, sem)

    def g2_wait(buf, sem):
        pltpu.make_async_copy(emb_hbm.at[gids_v.at[pl.ds(0, _C)]], buf,
                              sem).wait()

    def st_start(i, buf, sem):
        pltpu.async_copy(buf, out_hbm.at[pl.ds(base + i * _C, _C)], sem)

    def st_wait(buf, sem):
        pltpu.make_async_copy(buf, out_hbm.at[pl.ds(base, _C)], sem).wait()

    g2_start(0, r0, sg0)

    def pair(j, carry):
        i0 = 2 * j

        @pl.when(j > 0)
        def _():
            st_wait(r1, ss1)
        g2_start(i0 + 1, r1, sg1)
        g2_wait(r0, sg0)
        st_start(i0, r0, ss0)

        @pl.when(j < _NPAIR - 1)
        def _():
            st_wait(r0, ss0)
            g2_start(i0 + 2, r0, sg0)
        g2_wait(r1, sg1)
        st_start(i0 + 1, r1, ss1)
        return carry

    lax.fori_loop(0, _NPAIR, pair, 0)
    st_wait(r0, ss0)
    st_wait(r1, ss1)


def kernel(client_ids, item_ids, item_id2graph_id, item_embeddings):
    del client_ids
    ids_flat = item_ids.T.reshape(_N).astype(jnp.int32)
    out = _double_gather(ids_flat, item_id2graph_id.astype(jnp.int32),
                         item_embeddings)
    return out.reshape(_SEQ, _BATCH, _D).transpose(1, 0, 2)
